# Initial kernel scaffold; baseline (speedup 1.0000x reference)
import jax
import jax.numpy as jnp
from jax.experimental import pallas as pl


def kernel(x, x_mask, W_in, b_in, codebook, W_out, b_out):
    raise NotImplementedError("write your pallas kernel here")


# speedup vs baseline: 1.1648x; 1.1648x over previous
import functools

import jax
import jax.numpy as jnp
from jax import lax
from jax.experimental import pallas as pl
from jax.experimental.pallas import tpu as pltpu
from jax.experimental.pallas import tpu_sc as plsc

B, C_IN, T = 16, 256, 2048
DVQ, K = 32, 8192
BLK = 256
NSTEP = (B * T) // BLK
NT = T // BLK

_PREC = lax.Precision.DEFAULT


def _encode_body(x_ref, w_in_ref, b_in_ref, cb_ref, idx_ref, loss_ref):
    g = pl.program_id(0)
    xb = x_ref[0]
    z = lax.dot_general(w_in_ref[...], xb,
                        (((1,), (0,)), ((), ())), precision=_PREC)
    z = z + b_in_ref[...]
    cb = cb_ref[...]
    s = lax.dot_general(cb, z,
                        (((1,), (0,)), ((), ())), precision=_PREC)
    z2 = jnp.sum(z * z, axis=0, keepdims=True)
    e2 = jnp.sum(cb * cb, axis=1, keepdims=True)
    dist = (z2 - 2.0 * s) + e2
    mind = jnp.min(dist, axis=0, keepdims=True)
    ids = lax.broadcasted_iota(jnp.int32, dist.shape, 0)
    idx = jnp.min(jnp.where(dist == mind, ids, K), axis=0, keepdims=True)
    idx_ref[...] = idx[None]
    part = jnp.sum(mind)
    prev = jnp.where(g == 0, 0.0, loss_ref[0, 0])
    loss_ref[0, 0] = prev + part


def _decode_body(q_ref, w_out_ref, b_out_ref, mask_ref, out_ref):
    qb = q_ref[0][:, :DVQ]
    out = lax.dot_general(w_out_ref[...], qb,
                          (((1,), (1,)), ((), ())), precision=_PREC)
    out_ref[0] = (out + b_out_ref[...]) * mask_ref[0]


_NC, _NS = 2, 16
_NW = _NC * _NS
_ROWS_W = (B * T) // _NW
_CHUNK = 128
_NCH = _ROWS_W // _CHUNK


@functools.cache
def _build_gather():
    mesh = plsc.VectorSubcoreMesh(core_axis_name="c", subcore_axis_name="s",
                                  num_cores=_NC, num_subcores=_NS)

    @functools.partial(
        pl.kernel,
        mesh=mesh,
        out_type=jax.ShapeDtypeStruct((B * T, 128), jnp.float32),
        scratch_types=[
            pltpu.VMEM((_NCH, _CHUNK), jnp.int32),
            pltpu.VMEM((2, _CHUNK, 128), jnp.float32),
            pltpu.SemaphoreType.DMA,
            pltpu.SemaphoreType.DMA,
        ],
    )
    def _gather_body(cb_hbm, idx_hbm, q_hbm, idx_v, rows_v, sem0, sem1):
        wid = lax.axis_index("s") * _NC + lax.axis_index("c")
        sems = (sem0, sem1)
        pltpu.sync_copy(idx_hbm.at[pl.ds(wid * _NCH, _NCH)], idx_v)
        copies = [
            pltpu.async_copy(cb_hbm.at[idx_v.at[j]], rows_v.at[j % 2],
                             sems[j % 2])
            for j in range(2)
        ]
        for j in range(_NCH):
            copies[j].wait()
            pltpu.sync_copy(
                rows_v.at[j % 2],
                q_hbm.at[pl.ds(wid * _ROWS_W + j * _CHUNK, _CHUNK)])
            if j + 2 < _NCH:
                copies.append(
                    pltpu.async_copy(cb_hbm.at[idx_v.at[j + 2]],
                                     rows_v.at[j % 2], sems[j % 2]))

    return _gather_body


def _gather_codebook(codebook, idx2):
    cb_pad = jnp.pad(codebook, ((0, 0), (0, 128 - DVQ)))
    return _build_gather()(cb_pad, idx2)


def _encode(x, W_in, b_in, codebook):
    return pl.pallas_call(
        _encode_body,
        grid=(NSTEP,),
        in_specs=[
            pl.BlockSpec((1, C_IN, BLK), lambda g: (g // NT, 0, g % NT)),
            pl.BlockSpec((DVQ, C_IN), lambda g: (0, 0)),
            pl.BlockSpec((DVQ, 1), lambda g: (0, 0)),
            pl.BlockSpec((K, DVQ), lambda g: (0, 0)),
        ],
        out_specs=[
            pl.BlockSpec((1, 1, BLK), lambda g: (g, 0, 0)),
            pl.BlockSpec(memory_space=pltpu.SMEM, block_shape=(1, 1),
                         index_map=lambda g: (0, 0)),
        ],
        out_shape=[
            jax.ShapeDtypeStruct((NSTEP, 1, BLK), jnp.int32),
            jax.ShapeDtypeStruct((1, 1), jnp.float32),
        ],
        compiler_params=pltpu.CompilerParams(
            dimension_semantics=("arbitrary",)),
    )(x, W_in, b_in.reshape(DVQ, 1), codebook)


def _decode(q, W_out, b_out, x_mask):
    return pl.pallas_call(
        _decode_body,
        grid=(NSTEP,),
        in_specs=[
            pl.BlockSpec((1, BLK, 128), lambda g: (g, 0, 0)),
            pl.BlockSpec((C_IN, DVQ), lambda g: (0, 0)),
            pl.BlockSpec((C_IN, 1), lambda g: (0, 0)),
            pl.BlockSpec((1, 1, BLK), lambda g: (g // NT, 0, g % NT)),
        ],
        out_specs=pl.BlockSpec((1, C_IN, BLK), lambda g: (g // NT, 0, g % NT)),
        out_shape=jax.ShapeDtypeStruct((B, C_IN, T), jnp.float32),
        compiler_params=pltpu.CompilerParams(
            dimension_semantics=("arbitrary",)),
    )(q.reshape(NSTEP, BLK, 128), W_out, b_out.reshape(C_IN, 1), x_mask)


def kernel(x, x_mask, W_in, b_in, codebook, W_out, b_out):
    idx2, loss_sum = _encode(x, W_in, b_in, codebook)
    indices = idx2.reshape(B, T)
    q = _gather_codebook(codebook, idx2.reshape(_NW * _NCH, _CHUNK))
    out = _decode(q, W_out, b_out, x_mask)
    loss = loss_sum[0, 0] / jnp.float32(B * T * DVQ)
    return (out, indices, loss)


# speedup vs baseline: 1.1842x; 1.0167x over previous
import functools

import jax
import jax.numpy as jnp
from jax import lax
from jax.experimental import pallas as pl
from jax.experimental.pallas import tpu as pltpu
from jax.experimental.pallas import tpu_sc as plsc

B, C_IN, T = 16, 256, 2048
DVQ, K = 32, 8192
BLK = 256
NSTEP = (B * T) // BLK
NT = T // BLK

_PREC = lax.Precision.DEFAULT


def _encode_body(x_ref, w_in_ref, b_in_ref, cb_ref, idx_ref, loss_ref,
                 e2_ref, ids_ref):
    g = pl.program_id(0)
    xb = x_ref[0]
    z = lax.dot_general(w_in_ref[...], xb,
                        (((1,), (0,)), ((), ())), precision=_PREC)
    z = z + b_in_ref[...]
    cb = cb_ref[...]

    @pl.when(g == 0)
    def _():
        e2 = jnp.sum(cb * cb, axis=1, keepdims=True)
        e2_ref[...] = jnp.broadcast_to(e2, (K, BLK))
        ids_ref[...] = lax.broadcasted_iota(
            jnp.int32, (K, BLK), 0).astype(jnp.float32)

    s2 = lax.dot_general(cb, z + z,
                         (((1,), (0,)), ((), ())), precision=_PREC)
    z2 = jnp.sum(z * z, axis=0, keepdims=True)
    dist = (z2 - s2) + e2_ref[...]
    mind = jnp.min(dist, axis=0, keepdims=True)
    idxf = jnp.min(jnp.where(dist == mind, ids_ref[...], float(K)), axis=0,
                   keepdims=True)
    idx_ref[...] = idxf.astype(jnp.int32)[None]
    part = jnp.sum(mind)
    prev = jnp.where(g == 0, 0.0, loss_ref[0, 0])
    loss_ref[0, 0] = prev + part


def _decode_body(q_ref, w_out_ref, b_out_ref, mask_ref, out_ref):
    qb = q_ref[0][:, :DVQ]
    out = lax.dot_general(w_out_ref[...], qb,
                          (((1,), (1,)), ((), ())), precision=_PREC)
    out_ref[0] = (out + b_out_ref[...]) * mask_ref[0]


_NC, _NS = 2, 16
_NW = _NC * _NS
_ROWS_W = (B * T) // _NW
_CHUNK = 128
_NCH = _ROWS_W // _CHUNK


@functools.cache
def _build_gather():
    mesh = plsc.VectorSubcoreMesh(core_axis_name="c", subcore_axis_name="s",
                                  num_cores=_NC, num_subcores=_NS)

    @functools.partial(
        pl.kernel,
        mesh=mesh,
        out_type=jax.ShapeDtypeStruct((B * T, 128), jnp.float32),
        scratch_types=[
            pltpu.VMEM((_NCH, _CHUNK), jnp.int32),
            pltpu.VMEM((2, _CHUNK, 128), jnp.float32),
            pltpu.SemaphoreType.DMA,
            pltpu.SemaphoreType.DMA,
        ],
    )
    def _gather_body(cb_hbm, idx_hbm, q_hbm, idx_v, rows_v, sem0, sem1):
        wid = lax.axis_index("s") * _NC + lax.axis_index("c")
        sems = (sem0, sem1)
        pltpu.sync_copy(idx_hbm.at[pl.ds(wid * _NCH, _NCH)], idx_v)
        copies = [
            pltpu.async_copy(cb_hbm.at[idx_v.at[j]], rows_v.at[j % 2],
                             sems[j % 2])
            for j in range(2)
        ]
        for j in range(_NCH):
            copies[j].wait()
            pltpu.sync_copy(
                rows_v.at[j % 2],
                q_hbm.at[pl.ds(wid * _ROWS_W + j * _CHUNK, _CHUNK)])
            if j + 2 < _NCH:
                copies.append(
                    pltpu.async_copy(cb_hbm.at[idx_v.at[j + 2]],
                                     rows_v.at[j % 2], sems[j % 2]))

    return _gather_body


def _gather_codebook(codebook, idx2):
    cb_pad = jnp.pad(codebook, ((0, 0), (0, 128 - DVQ)))
    return _build_gather()(cb_pad, idx2)


def _encode(x, W_in, b_in, codebook):
    return pl.pallas_call(
        _encode_body,
        grid=(NSTEP,),
        in_specs=[
            pl.BlockSpec((1, C_IN, BLK), lambda g: (g // NT, 0, g % NT)),
            pl.BlockSpec((DVQ, C_IN), lambda g: (0, 0)),
            pl.BlockSpec((DVQ, 1), lambda g: (0, 0)),
            pl.BlockSpec((K, DVQ), lambda g: (0, 0)),
        ],
        out_specs=[
            pl.BlockSpec((1, 1, BLK), lambda g: (g, 0, 0)),
            pl.BlockSpec(memory_space=pltpu.SMEM, block_shape=(1, 1),
                         index_map=lambda g: (0, 0)),
        ],
        out_shape=[
            jax.ShapeDtypeStruct((NSTEP, 1, BLK), jnp.int32),
            jax.ShapeDtypeStruct((1, 1), jnp.float32),
        ],
        scratch_shapes=[pltpu.VMEM((K, BLK), jnp.float32),
                        pltpu.VMEM((K, BLK), jnp.float32)],
        compiler_params=pltpu.CompilerParams(
            dimension_semantics=("arbitrary",)),
    )(x, W_in, b_in.reshape(DVQ, 1), codebook)


def _decode(q, W_out, b_out, x_mask):
    return pl.pallas_call(
        _decode_body,
        grid=(NSTEP,),
        in_specs=[
            pl.BlockSpec((1, BLK, 128), lambda g: (g, 0, 0)),
            pl.BlockSpec((C_IN, DVQ), lambda g: (0, 0)),
            pl.BlockSpec((C_IN, 1), lambda g: (0, 0)),
            pl.BlockSpec((1, 1, BLK), lambda g: (g // NT, 0, g % NT)),
        ],
        out_specs=pl.BlockSpec((1, C_IN, BLK), lambda g: (g // NT, 0, g % NT)),
        out_shape=jax.ShapeDtypeStruct((B, C_IN, T), jnp.float32),
        compiler_params=pltpu.CompilerParams(
            dimension_semantics=("arbitrary",)),
    )(q.reshape(NSTEP, BLK, 128), W_out, b_out.reshape(C_IN, 1), x_mask)


def kernel(x, x_mask, W_in, b_in, codebook, W_out, b_out):
    idx2, loss_sum = _encode(x, W_in, b_in, codebook)
    indices = idx2.reshape(B, T)
    q = _gather_codebook(codebook, idx2.reshape(_NW * _NCH, _CHUNK))
    out = _decode(q, W_out, b_out, x_mask)
    loss = loss_sum[0, 0] / jnp.float32(B * T * DVQ)
    return (out, indices, loss)


# speedup vs baseline: 1.4348x; 1.2116x over previous
import functools

import jax
import jax.numpy as jnp
from jax import lax
from jax.experimental import pallas as pl
from jax.experimental.pallas import tpu as pltpu
from jax.experimental.pallas import tpu_sc as plsc

B, C_IN, T = 16, 256, 2048
DVQ, K = 32, 8192
BLK = 256
NSTEP = (B * T) // BLK
NT = T // BLK

_PREC = lax.Precision.DEFAULT


def _encode_body(x_ref, w_in_ref, b_in_ref, cb_ref, idx_ref, loss_ref,
                 e2_ref):
    g = pl.program_id(0)
    xb = x_ref[0]
    z = lax.dot_general(w_in_ref[...], xb,
                        (((1,), (0,)), ((), ())), precision=_PREC)
    z = z + b_in_ref[...]
    cb = cb_ref[...]

    @pl.when(g == 0)
    def _():
        e2 = jnp.sum(cb * cb, axis=1, keepdims=True)
        e2_ref[...] = jnp.broadcast_to(e2, (K, BLK))

    s2 = lax.dot_general(cb, z + z,
                         (((1,), (0,)), ((), ())), precision=_PREC)
    z2 = jnp.sum(z * z, axis=0, keepdims=True)
    dist = (z2 - s2) + e2_ref[...]
    mind = jnp.min(dist, axis=0, keepdims=True)
    idx = jnp.argmin(dist, axis=0).astype(jnp.int32)[None, None]
    idx_ref[...] = idx
    part = jnp.sum(mind)
    prev = jnp.where(g == 0, 0.0, loss_ref[0, 0])
    loss_ref[0, 0] = prev + part


def _decode_body(q_ref, w_out_ref, b_out_ref, mask_ref, out_ref):
    qb = q_ref[0][:, :DVQ]
    out = lax.dot_general(w_out_ref[...], qb,
                          (((1,), (1,)), ((), ())), precision=_PREC)
    out_ref[0] = (out + b_out_ref[...]) * mask_ref[0]


_NC, _NS = 2, 16
_NW = _NC * _NS
_ROWS_W = (B * T) // _NW
_CHUNK = 128
_NCH = _ROWS_W // _CHUNK


@functools.cache
def _build_gather():
    mesh = plsc.VectorSubcoreMesh(core_axis_name="c", subcore_axis_name="s",
                                  num_cores=_NC, num_subcores=_NS)

    @functools.partial(
        pl.kernel,
        mesh=mesh,
        out_type=jax.ShapeDtypeStruct((B * T, 128), jnp.float32),
        scratch_types=[
            pltpu.VMEM((_NCH, _CHUNK), jnp.int32),
            pltpu.VMEM((2, _CHUNK, 128), jnp.float32),
            pltpu.SemaphoreType.DMA,
            pltpu.SemaphoreType.DMA,
        ],
    )
    def _gather_body(cb_hbm, idx_hbm, q_hbm, idx_v, rows_v, sem0, sem1):
        wid = lax.axis_index("s") * _NC + lax.axis_index("c")
        sems = (sem0, sem1)
        pltpu.sync_copy(idx_hbm.at[pl.ds(wid * _NCH, _NCH)], idx_v)
        copies = [
            pltpu.async_copy(cb_hbm.at[idx_v.at[j]], rows_v.at[j % 2],
                             sems[j % 2])
            for j in range(2)
        ]
        for j in range(_NCH):
            copies[j].wait()
            pltpu.sync_copy(
                rows_v.at[j % 2],
                q_hbm.at[pl.ds(wid * _ROWS_W + j * _CHUNK, _CHUNK)])
            if j + 2 < _NCH:
                copies.append(
                    pltpu.async_copy(cb_hbm.at[idx_v.at[j + 2]],
                                     rows_v.at[j % 2], sems[j % 2]))

    return _gather_body


def _gather_codebook(codebook, idx2):
    cb_pad = jnp.pad(codebook, ((0, 0), (0, 128 - DVQ)))
    return _build_gather()(cb_pad, idx2)


def _encode(x, W_in, b_in, codebook):
    return pl.pallas_call(
        _encode_body,
        grid=(NSTEP,),
        in_specs=[
            pl.BlockSpec((1, C_IN, BLK), lambda g: (g // NT, 0, g % NT)),
            pl.BlockSpec((DVQ, C_IN), lambda g: (0, 0)),
            pl.BlockSpec((DVQ, 1), lambda g: (0, 0)),
            pl.BlockSpec((K, DVQ), lambda g: (0, 0)),
        ],
        out_specs=[
            pl.BlockSpec((1, 1, BLK), lambda g: (g, 0, 0)),
            pl.BlockSpec(memory_space=pltpu.SMEM, block_shape=(1, 1),
                         index_map=lambda g: (0, 0)),
        ],
        out_shape=[
            jax.ShapeDtypeStruct((NSTEP, 1, BLK), jnp.int32),
            jax.ShapeDtypeStruct((1, 1), jnp.float32),
        ],
        scratch_shapes=[pltpu.VMEM((K, BLK), jnp.float32)],
        compiler_params=pltpu.CompilerParams(
            dimension_semantics=("arbitrary",)),
    )(x, W_in, b_in.reshape(DVQ, 1), codebook)


def _decode(q, W_out, b_out, x_mask):
    return pl.pallas_call(
        _decode_body,
        grid=(NSTEP,),
        in_specs=[
            pl.BlockSpec((1, BLK, 128), lambda g: (g, 0, 0)),
            pl.BlockSpec((C_IN, DVQ), lambda g: (0, 0)),
            pl.BlockSpec((C_IN, 1), lambda g: (0, 0)),
            pl.BlockSpec((1, 1, BLK), lambda g: (g // NT, 0, g % NT)),
        ],
        out_specs=pl.BlockSpec((1, C_IN, BLK), lambda g: (g // NT, 0, g % NT)),
        out_shape=jax.ShapeDtypeStruct((B, C_IN, T), jnp.float32),
        compiler_params=pltpu.CompilerParams(
            dimension_semantics=("arbitrary",)),
    )(q.reshape(NSTEP, BLK, 128), W_out, b_out.reshape(C_IN, 1), x_mask)


def kernel(x, x_mask, W_in, b_in, codebook, W_out, b_out):
    idx2, loss_sum = _encode(x, W_in, b_in, codebook)
    indices = idx2.reshape(B, T)
    q = _gather_codebook(codebook, idx2.reshape(_NW * _NCH, _CHUNK))
    out = _decode(q, W_out, b_out, x_mask)
    loss = loss_sum[0, 0] / jnp.float32(B * T * DVQ)
    return (out, indices, loss)


# speedup vs baseline: 1.5410x; 1.0740x over previous
import functools

import jax
import jax.numpy as jnp
from jax import lax
from jax.experimental import pallas as pl
from jax.experimental.pallas import tpu as pltpu
from jax.experimental.pallas import tpu_sc as plsc

B, C_IN, T = 16, 256, 2048
DVQ, K = 32, 8192
BLK = 256
NSTEP = (B * T) // BLK
NT = T // BLK

_PREC = lax.Precision.DEFAULT


def _e2_body(cb_ref, e2_ref):
    cb = cb_ref[...]
    e2 = jnp.sum(cb * cb, axis=1, keepdims=True)
    e2_ref[...] = jnp.broadcast_to(e2, (K, BLK))


def _encode_body(x_ref, w_in_ref, b_in_ref, cb_ref, e2_ref, idx_ref, loss_ref):
    g = pl.program_id(0)
    xb = x_ref[0]
    z = lax.dot_general(w_in_ref[...], xb,
                        (((1,), (0,)), ((), ())), precision=_PREC)
    z = z + b_in_ref[...]
    cb = cb_ref[...]
    s2 = lax.dot_general(cb, z + z,
                         (((1,), (0,)), ((), ())), precision=_PREC)
    z2 = jnp.sum(z * z, axis=0, keepdims=True)
    dist = (z2 - s2) + e2_ref[...]
    mind = jnp.min(dist, axis=0, keepdims=True)
    idx = jnp.argmin(dist, axis=0).astype(jnp.int32)[None, None]
    idx_ref[...] = idx
    part = jnp.sum(mind)
    prev = jnp.where(g == 0, 0.0, loss_ref[0, 0])
    loss_ref[0, 0] = prev + part


def _decode_body(q_ref, w_out_ref, b_out_ref, mask_ref, out_ref):
    qb = q_ref[0][:, :DVQ]
    out = lax.dot_general(w_out_ref[...], qb,
                          (((1,), (1,)), ((), ())), precision=_PREC)
    out_ref[0] = (out + b_out_ref[...]) * mask_ref[0]


_NC, _NS = 2, 16
_NW = _NC * _NS
_ROWS_W = (B * T) // _NW
_CHUNK = 128
_NCH = _ROWS_W // _CHUNK


@functools.cache
def _build_gather():
    mesh = plsc.VectorSubcoreMesh(core_axis_name="c", subcore_axis_name="s",
                                  num_cores=_NC, num_subcores=_NS)

    @functools.partial(
        pl.kernel,
        mesh=mesh,
        out_type=jax.ShapeDtypeStruct((B * T, 128), jnp.float32),
        scratch_types=[
            pltpu.VMEM((_NCH, _CHUNK), jnp.int32),
            pltpu.VMEM((2, _CHUNK, 128), jnp.float32),
            pltpu.SemaphoreType.DMA,
            pltpu.SemaphoreType.DMA,
        ],
    )
    def _gather_body(cb_hbm, idx_hbm, q_hbm, idx_v, rows_v, sem0, sem1):
        wid = lax.axis_index("s") * _NC + lax.axis_index("c")
        sems = (sem0, sem1)
        pltpu.sync_copy(idx_hbm.at[pl.ds(wid * _NCH, _NCH)], idx_v)
        copies = [
            pltpu.async_copy(cb_hbm.at[idx_v.at[j]], rows_v.at[j % 2],
                             sems[j % 2])
            for j in range(2)
        ]
        for j in range(_NCH):
            copies[j].wait()
            pltpu.sync_copy(
                rows_v.at[j % 2],
                q_hbm.at[pl.ds(wid * _ROWS_W + j * _CHUNK, _CHUNK)])
            if j + 2 < _NCH:
                copies.append(
                    pltpu.async_copy(cb_hbm.at[idx_v.at[j + 2]],
                                     rows_v.at[j % 2], sems[j % 2]))

    return _gather_body


def _gather_codebook(codebook, idx2):
    cb_pad = jnp.pad(codebook, ((0, 0), (0, 128 - DVQ)))
    return _build_gather()(cb_pad, idx2)


def _encode(x, W_in, b_in, codebook):
    e2b = pl.pallas_call(
        _e2_body,
        in_specs=[pl.BlockSpec((K, DVQ), lambda: (0, 0))],
        out_specs=pl.BlockSpec((K, BLK), lambda: (0, 0)),
        out_shape=jax.ShapeDtypeStruct((K, BLK), jnp.float32),
    )(codebook)
    return pl.pallas_call(
        _encode_body,
        grid=(NSTEP,),
        in_specs=[
            pl.BlockSpec((1, C_IN, BLK), lambda g: (g // NT, 0, g % NT)),
            pl.BlockSpec((DVQ, C_IN), lambda g: (0, 0)),
            pl.BlockSpec((DVQ, 1), lambda g: (0, 0)),
            pl.BlockSpec((K, DVQ), lambda g: (0, 0)),
            pl.BlockSpec((K, BLK), lambda g: (0, 0)),
        ],
        out_specs=[
            pl.BlockSpec((1, 1, BLK), lambda g: (g, 0, 0)),
            pl.BlockSpec(memory_space=pltpu.SMEM, block_shape=(1, 1),
                         index_map=lambda g: (0, 0)),
        ],
        out_shape=[
            jax.ShapeDtypeStruct((NSTEP, 1, BLK), jnp.int32),
            jax.ShapeDtypeStruct((1, 1), jnp.float32),
        ],
        compiler_params=pltpu.CompilerParams(
            dimension_semantics=("arbitrary",)),
    )(x, W_in, b_in.reshape(DVQ, 1), codebook, e2b)


def _decode(q, W_out, b_out, x_mask):
    return pl.pallas_call(
        _decode_body,
        grid=(NSTEP,),
        in_specs=[
            pl.BlockSpec((1, BLK, 128), lambda g: (g, 0, 0)),
            pl.BlockSpec((C_IN, DVQ), lambda g: (0, 0)),
            pl.BlockSpec((C_IN, 1), lambda g: (0, 0)),
            pl.BlockSpec((1, 1, BLK), lambda g: (g // NT, 0, g % NT)),
        ],
        out_specs=pl.BlockSpec((1, C_IN, BLK), lambda g: (g // NT, 0, g % NT)),
        out_shape=jax.ShapeDtypeStruct((B, C_IN, T), jnp.float32),
        compiler_params=pltpu.CompilerParams(
            dimension_semantics=("arbitrary",)),
    )(q.reshape(NSTEP, BLK, 128), W_out, b_out.reshape(C_IN, 1), x_mask)


def kernel(x, x_mask, W_in, b_in, codebook, W_out, b_out):
    idx2, loss_sum = _encode(x, W_in, b_in, codebook)
    indices = idx2.reshape(B, T)
    q = _gather_codebook(codebook, idx2.reshape(_NW * _NCH, _CHUNK))
    out = _decode(q, W_out, b_out, x_mask)
    loss = loss_sum[0, 0] / jnp.float32(B * T * DVQ)
    return (out, indices, loss)
